# baseline (speedup 1.0000x reference)
import functools

import jax
import jax.numpy as jnp
from jax import lax
from jax.experimental import pallas as pl
from jax.experimental.pallas import tpu as pltpu
from jax.experimental.pallas import tpu_sc as plsc

NUM_EMB = 11
HIDDEN = 512
LANES = 16
NC = 2
NS = 16
NW = NC * NS

GB, GR, GM = 256, 30, 30
B = GB * GR * GM
BPW = B // NW
BPERW = GB // NW
PCHUNK = 3
CROWS = PCHUNK * GM
NCHUNK = BPW // CROWS
HGM = GM // 2

_MESH = plsc.VectorSubcoreMesh(
    core_axis_name="c", subcore_axis_name="s", num_cores=NC, num_subcores=NS
)


@functools.partial(
    pl.kernel,
    out_type=jax.ShapeDtypeStruct((GB, GR, GM, HIDDEN), jnp.float32),
    mesh=_MESH,
    compiler_params=pltpu.CompilerParams(
        needs_layout_passes=False, use_tc_tiling_on_sc=True
    ),
    scratch_types=[
        pltpu.VMEM((BPW + LANES,), jnp.int32),
        pltpu.VMEM((NUM_EMB, HIDDEN), jnp.float32),
        pltpu.VMEM((PCHUNK, GM, HIDDEN), jnp.float32),
        pltpu.VMEM((PCHUNK, GM, HIDDEN), jnp.float32),
        pltpu.SemaphoreType.DMA,
        pltpu.SemaphoreType.DMA,
    ],
)
def _emb_lookup(table_hbm, idx_hbm, out_hbm, idx_v, table_v, rows0, rows1,
                osem0, osem1):
    wid = lax.axis_index("s") * NC + lax.axis_index("c")
    base = pl.multiple_of(wid * BPW, BPW)

    rows = (rows0, rows1)
    osem = (osem0, osem1)

    pltpu.sync_copy(table_hbm, table_v)
    pltpu.sync_copy(idx_hbm.at[pl.ds(base, BPW)], idx_v.at[pl.ds(0, BPW)])

    iota = jnp.arange(LANES, dtype=jnp.int32)
    pad = jnp.full((LANES,), NUM_EMB - 1, jnp.int32)
    colv = [jnp.bitwise_and(iota + s, LANES - 1) for s in range(LANES)]

    def build(i, buf):
        @pl.loop(0, CROWS // HGM)
        def _(rg):
            v16 = idx_v[pl.ds(i * CROWS + rg * HGM, LANES)]
            v16 = jnp.where(v16 == -1, pad, v16)
            vs = [v16[j] for j in range(HGM)]
            plane = rg // 2
            prow0 = (rg - plane * 2) * HGM

            @plsc.parallel_loop(0, HIDDEN // LANES, unroll=2)
            def _(g):
                gc = g * LANES
                for j in range(HGM):
                    buf[plane, prow0 + j, pl.ds(gc, LANES)] = (
                        table_v[vs[j], pl.ds(gc, LANES)])

    def put(i, b):
        bb = i // (GR // PCHUNK)
        r0 = (i - bb * (GR // PCHUNK)) * PCHUNK
        return pltpu.make_async_copy(
            rows[b],
            out_hbm.at[BPERW * wid + bb, pl.ds(r0, PCHUNK)],
            osem[b],
        )

    build(0, rows[0])
    put(0, 0).start()
    build(1, rows[1])
    put(1, 1).start()

    @pl.loop(1, NCHUNK // 2)
    def _(k):
        i0 = k * 2
        put(i0, 0).wait()
        build(i0, rows[0])
        put(i0, 0).start()
        put(i0 + 1, 1).wait()
        build(i0 + 1, rows[1])
        put(i0 + 1, 1).start()

    put(NCHUNK - 2, 0).wait()
    put(NCHUNK - 1, 1).wait()


def kernel(grid, table):
    idx = grid.reshape(-1).astype(jnp.int32)
    return _emb_lookup(table, idx)


# speedup vs baseline: 1.1180x; 1.1180x over previous
import functools

import jax
import jax.numpy as jnp
from jax import lax
from jax.experimental import pallas as pl
from jax.experimental.pallas import tpu as pltpu
from jax.experimental.pallas import tpu_sc as plsc

NUM_EMB = 11
HIDDEN = 512
LANES = 16
NC = 2
NS = 16
NW = NC * NS

GB, GR, GM = 256, 30, 30
B = GB * GR * GM
BPW = B // NW
BPERW = GB // NW
PCHUNK = 3
CROWS = PCHUNK * GM
NCHUNK = BPW // CROWS
HGM = GM // 2

_MESH = plsc.VectorSubcoreMesh(
    core_axis_name="c", subcore_axis_name="s", num_cores=NC, num_subcores=NS
)


@functools.partial(
    pl.kernel,
    out_type=jax.ShapeDtypeStruct((GB, GR, GM, HIDDEN), jnp.float32),
    mesh=_MESH,
    compiler_params=pltpu.CompilerParams(
        needs_layout_passes=False, use_tc_tiling_on_sc=True
    ),
    scratch_types=[
        pltpu.VMEM((BPW + LANES,), jnp.int32),
        pltpu.VMEM((NUM_EMB, HIDDEN), jnp.float32),
        pltpu.VMEM((PCHUNK, GM, HIDDEN), jnp.float32),
        pltpu.VMEM((PCHUNK, GM, HIDDEN), jnp.float32),
        pltpu.SemaphoreType.DMA,
        pltpu.SemaphoreType.DMA,
    ],
)
def _emb_lookup(table_hbm, idx_hbm, out_hbm, idx_v, table_v, rows0, rows1,
                osem0, osem1):
    wid = lax.axis_index("s") * NC + lax.axis_index("c")
    base = pl.multiple_of(wid * BPW, BPW)

    rows = (rows0, rows1)
    osem = (osem0, osem1)

    pltpu.sync_copy(table_hbm, table_v)
    pltpu.sync_copy(idx_hbm.at[pl.ds(base, BPW)], idx_v.at[pl.ds(0, BPW)])

    iota = jnp.arange(LANES, dtype=jnp.int32)
    pad = jnp.full((LANES,), NUM_EMB - 1, jnp.int32)
    colv = [jnp.bitwise_and(iota + s, LANES - 1) for s in range(LANES)]

    def build(i, buf):
        @pl.loop(0, PCHUNK)
        def _(rg):
            b0 = i * CROWS + rg * GM
            va = idx_v[pl.ds(b0, LANES)]
            vb = idx_v[pl.ds(b0 + LANES, LANES)]
            va = jnp.where(va == -1, pad, va)
            vb = jnp.where(vb == -1, pad, vb)
            vs = [va[j] for j in range(LANES)] + [vb[j] for j in range(GM - LANES)]

            @plsc.parallel_loop(0, HIDDEN // LANES, unroll=4)
            def _(g):
                gc = g * LANES
                for j in range(GM):
                    buf[rg, j, pl.ds(gc, LANES)] = table_v[vs[j], pl.ds(gc, LANES)]

    def put(i, b):
        bb = i // (GR // PCHUNK)
        r0 = (i - bb * (GR // PCHUNK)) * PCHUNK
        return pltpu.make_async_copy(
            rows[b],
            out_hbm.at[BPERW * wid + bb, pl.ds(r0, PCHUNK)],
            osem[b],
        )

    build(0, rows[0])
    put(0, 0).start()
    build(1, rows[1])
    put(1, 1).start()

    @pl.loop(1, NCHUNK // 2)
    def _(k):
        i0 = k * 2
        put(i0, 0).wait()
        build(i0, rows[0])
        put(i0, 0).start()
        put(i0 + 1, 1).wait()
        build(i0 + 1, rows[1])
        put(i0 + 1, 1).start()

    put(NCHUNK - 2, 0).wait()
    put(NCHUNK - 1, 1).wait()


def kernel(grid, table):
    idx = grid.reshape(-1).astype(jnp.int32)
    return _emb_lookup(table, idx)
